# baseline (speedup 1.0000x reference)
import functools

import jax
import jax.numpy as jnp
from jax import lax
from jax.experimental import pallas as pl
from jax.experimental.pallas import tpu as pltpu
from jax.experimental.pallas import tpu_sc as plsc

B, L, DIM, K = 1024, 20, 128, 1000
KP = 1024
EP = 1008
NW = 32
RPW = (B * L) // NW
CHUNK = 64
NCHUNK = RPW // CHUNK
BPW = B // NW
BBLK = 64
KPH = 512
PBLK = 128
F32 = jnp.float32


def _tables_dense_body(emb_ref, w5t_ref, corr_ref, data_ref, w1t_ref,
                       w2t_ref, w3t_ref, w4t_ref, abcd_ref, t_ref, d_ref):
    t = jnp.tanh(jnp.dot(emb_ref[...], w5t_ref[...],
                         preferred_element_type=F32))
    tb = (t * corr_ref[...]).astype(jnp.bfloat16)
    lo = jax.lax.bitcast_convert_type(tb[:, :KPH], jnp.uint16).astype(jnp.uint32)
    hi = jax.lax.bitcast_convert_type(tb[:, KPH:], jnp.uint16).astype(jnp.uint32)
    t_ref[...] = jax.lax.bitcast_convert_type(lo | (hi << 16), jnp.int32)
    dmean = jnp.sum(data_ref[...], axis=0) * (1.0 / L)
    acc = jnp.zeros((B, KP), F32)
    for i, wref in enumerate((w1t_ref, w2t_ref, w3t_ref, w4t_ref)):
        p = jnp.dot(dmean, wref[...], preferred_element_type=F32)
        n = jnp.sqrt(jnp.sum(p * p, axis=1, keepdims=True))
        acc = acc + (p / jnp.maximum(n, 1e-5)) * abcd_ref[i]
    d_ref[...] = acc


def _sc_body(idx3_hbm, idxf_hbm, t_hbm, g_hbm, tgt_hbm,
             idx_v, ev_v, rows_a, rows_b, tgt_v, sem,
             gsem_a, gsem_b, wsem_a, wsem_b):
    wid = lax.axis_index("c") * 16 + lax.axis_index("s")
    pltpu.sync_copy(idx3_hbm.at[wid], idx_v)
    bufs = (rows_a, rows_b)
    gsems = (gsem_a, gsem_b)
    wsems = (wsem_a, wsem_b)

    def _out(c):
        return g_hbm.at[pl.ds(wid * RPW + c * CHUNK, CHUNK), :]

    gd = pltpu.async_copy(t_hbm.at[idx_v.at[0]], bufs[0], gsems[0])
    wd = [None, None]
    for c in range(1, NCHUNK):
        b = c % 2
        if wd[b] is not None:
            wd[b].wait()
        gd_next = pltpu.async_copy(t_hbm.at[idx_v.at[c]], bufs[b], gsems[b])
        gd.wait()
        wd[1 - b] = pltpu.async_copy(bufs[1 - b], _out(c - 1), wsems[1 - b])
        gd = gd_next
    gd.wait()
    lastb = (NCHUNK - 1) % 2
    wd[lastb] = pltpu.async_copy(bufs[lastb], _out(NCHUNK - 1), wsems[lastb])
    wd[0].wait()
    wd[1].wait()
    pltpu.sync_copy(idxf_hbm.at[pl.ds(wid * RPW, RPW)], ev_v)
    ones16 = jnp.full((16,), 1.0, F32)

    def _memset(i, carry):
        tgt_v[pl.ds(pl.multiple_of(i * 16, 16), 16)] = ones16
        return carry

    lax.fori_loop(0, (BPW * KP) // 16, _memset, 0)
    zeros16 = jnp.zeros((16,), F32)
    lanes = lax.iota(jnp.int32, 16)
    for j in range(RPW // 16):
        e16 = ev_v[pl.ds(j * 16, 16)]
        bloc = (lanes + j * 16) // L
        kidx = jnp.where(e16 != 0, e16 - 1, jnp.int32(K))
        plsc.store_scatter(tgt_v, [bloc * KP + kidx], zeros16)
    pltpu.sync_copy(tgt_v, tgt_hbm.at[pl.ds(wid * (BPW * KP), BPW * KP)])


NWRD = K // 8


def _pack_body(du_ref, p_ref):
    kmod = jax.lax.broadcasted_iota(jnp.int32, (K, B), 0) % 8
    wt = jnp.exp2(kmod.astype(F32))
    sel = jnp.where(du_ref[0] >= 0.5, wt, 0.0)
    w = jnp.sum(sel.reshape(NWRD, 8, B), axis=1)
    p_ref[0] = w.astype(jnp.int32)


def _combine_body(pt_ref, g_ref, d_ref, o_ref):
    wi = jax.lax.broadcasted_iota(jnp.int32, (NWRD, KP), 0)
    ki = jax.lax.broadcasted_iota(jnp.int32, (NWRD, KP), 1)
    ex = jnp.where(ki // 8 == wi, jnp.exp2(-(ki % 8).astype(F32)), 0.0)
    exb = ex.astype(jnp.bfloat16)
    acc_lo = jnp.zeros((BBLK, KPH), F32)
    acc_hi = jnp.zeros((BBLK, K - KPH), F32)
    for l in range(L):
        g32 = g_ref[l]
        g_lo = jax.lax.bitcast_convert_type(g32 << 16, F32)
        g_hi = jax.lax.bitcast_convert_type(
            g32 & jnp.int32(-65536), F32)[:, :K - KPH]
        wf = pt_ref[:, l, :].astype(jnp.bfloat16)
        e = jnp.dot(wf, exb, preferred_element_type=F32)
        bf = (e.astype(jnp.int32) & 1).astype(F32)
        acc_lo = acc_lo + g_lo * bf[:, :KPH]
        acc_hi = acc_hi + g_hi * bf[:, KPH:K]
    item_lo = acc_lo * 2.0
    item_hi = acc_hi * 2.0
    n2 = (jnp.sum(item_lo * item_lo, axis=1, keepdims=True)
          + jnp.sum(item_hi * item_hi, axis=1, keepdims=True))
    inv = 1.0 / jnp.maximum(jnp.sqrt(n2), 1e-5)
    o_ref[:, :KPH] = jnp.tanh(d_ref[:, :KPH] + item_lo * inv)
    o_ref[:, KPH:] = jnp.tanh(d_ref[:, KPH:K] + item_hi * inv)


def kernel(data, event_type, place_correlation, dropout_u, emb_table,
           W5, W1, W2, W3, W4, a, b, c, d):
    emb_pad = jnp.pad(emb_table, ((0, EP - (K + 1)), (0, 0)))
    w5t = jnp.pad(W5, ((0, KP - K), (0, 0))).T
    w1t = jnp.pad(W1, ((0, KP - K), (0, 0))).T
    w2t = jnp.pad(W2, ((0, KP - K), (0, 0))).T
    w3t = jnp.pad(W3, ((0, KP - K), (0, 0))).T
    w4t = jnp.pad(W4, ((0, KP - K), (0, 0))).T
    corr_pad = jnp.pad(place_correlation, ((1, EP - K - 1), (0, KP - K)))
    abcd = jnp.concatenate([a, b, c, d])
    idx_flat = event_type.reshape(-1)
    idx3 = event_type.T.reshape(NW, NCHUNK, CHUNK)

    t_tab, dense = pl.pallas_call(
        _tables_dense_body,
        out_shape=(jax.ShapeDtypeStruct((EP, KPH), jnp.int32),
                   jax.ShapeDtypeStruct((B, KP), F32)),
        in_specs=[
            pl.BlockSpec((EP, DIM), lambda: (0, 0)),
            pl.BlockSpec((DIM, KP), lambda: (0, 0)),
            pl.BlockSpec((EP, KP), lambda: (0, 0)),
            pl.BlockSpec((L, B, DIM), lambda: (0, 0, 0)),
            pl.BlockSpec((DIM, KP), lambda: (0, 0)),
            pl.BlockSpec((DIM, KP), lambda: (0, 0)),
            pl.BlockSpec((DIM, KP), lambda: (0, 0)),
            pl.BlockSpec((DIM, KP), lambda: (0, 0)),
            pl.BlockSpec(memory_space=pltpu.SMEM),
        ],
        out_specs=(pl.BlockSpec((EP, KPH), lambda: (0, 0)),
                   pl.BlockSpec((B, KP), lambda: (0, 0))),
    )(emb_pad, w5t, corr_pad, jnp.transpose(data, (1, 0, 2)),
      w1t, w2t, w3t, w4t, abcd)

    mesh = plsc.VectorSubcoreMesh(core_axis_name="c", subcore_axis_name="s",
                                  num_cores=2, num_subcores=16)
    sc = functools.partial(
        pl.kernel,
        compiler_params=pltpu.CompilerParams(needs_layout_passes=False),
        out_type=(jax.ShapeDtypeStruct((B * L, KPH), jnp.int32),
                  jax.ShapeDtypeStruct((B * KP,), F32)),
        mesh=mesh,
        scratch_types=[
            pltpu.VMEM((NCHUNK, CHUNK), jnp.int32),
            pltpu.VMEM((RPW,), jnp.int32),
            pltpu.VMEM((CHUNK, KPH), jnp.int32),
            pltpu.VMEM((CHUNK, KPH), jnp.int32),
            pltpu.VMEM((BPW * KP,), F32),
            pltpu.SemaphoreType.DMA,
            pltpu.SemaphoreType.DMA,
            pltpu.SemaphoreType.DMA,
            pltpu.SemaphoreType.DMA,
            pltpu.SemaphoreType.DMA,
        ],
    )(_sc_body)
    g_rows, tgt_flat = sc(idx3, idx_flat, t_tab)

    packed = pl.pallas_call(
        _pack_body,
        grid=(L,),
        out_shape=jax.ShapeDtypeStruct((L, NWRD, B), jnp.int32),
        in_specs=[pl.BlockSpec((1, K, B), lambda i: (i, 0, 0))],
        out_specs=pl.BlockSpec((1, NWRD, B), lambda i: (i, 0, 0)),
    )(jnp.transpose(dropout_u, (1, 2, 0)))
    packed_t = jnp.transpose(packed, (2, 0, 1))

    out = pl.pallas_call(
        _combine_body,
        grid=(B // BBLK,),
        out_shape=jax.ShapeDtypeStruct((B, K), F32),
        in_specs=[
            pl.BlockSpec((BBLK, L, NWRD), lambda i: (i, 0, 0)),
            pl.BlockSpec((L, BBLK, KPH), lambda i: (0, i, 0)),
            pl.BlockSpec((BBLK, KP), lambda i: (i, 0)),
        ],
        out_specs=pl.BlockSpec((BBLK, K), lambda i: (i, 0)),
    )(packed_t, g_rows.reshape(L, B, KPH), dense)

    target = tgt_flat.reshape(B, KP)[:, :K]
    return out, target


# speedup vs baseline: 1.0797x; 1.0797x over previous
import functools

import jax
import jax.numpy as jnp
from jax import lax
from jax.experimental import pallas as pl
from jax.experimental.pallas import tpu as pltpu
from jax.experimental.pallas import tpu_sc as plsc

B, L, DIM, K = 1024, 20, 128, 1000
KP = 1024
EP = 1008
NW = 32
RPW = (B * L) // NW
CHUNK = 64
NCHUNK = RPW // CHUNK
BPW = B // NW
BBLK = 64
KPH = 512
PBLK = 128
F32 = jnp.float32


def _tables_dense_body(emb_ref, w5t_ref, corr_ref, data_ref, w1t_ref,
                       w2t_ref, w3t_ref, w4t_ref, abcd_ref, t_ref, d_ref):
    t = jnp.tanh(jnp.dot(emb_ref[...], w5t_ref[...],
                         preferred_element_type=F32))
    tb = (t * corr_ref[...]).astype(jnp.bfloat16)
    lo = jax.lax.bitcast_convert_type(tb[:, :KPH], jnp.uint16).astype(jnp.uint32)
    hi = jax.lax.bitcast_convert_type(tb[:, KPH:], jnp.uint16).astype(jnp.uint32)
    t_ref[...] = jax.lax.bitcast_convert_type(lo | (hi << 16), jnp.int32)
    dmean = jnp.sum(data_ref[...], axis=0) * (1.0 / L)
    acc = jnp.zeros((B, KP), F32)
    for i, wref in enumerate((w1t_ref, w2t_ref, w3t_ref, w4t_ref)):
        p = jnp.dot(dmean, wref[...], preferred_element_type=F32)
        n = jnp.sqrt(jnp.sum(p * p, axis=1, keepdims=True))
        acc = acc + (p / jnp.maximum(n, 1e-5)) * abcd_ref[i]
    d_ref[...] = acc


def _sc_body(idx3_hbm, idxf_hbm, t_hbm, g_hbm, tgt_hbm,
             idx_v, ev_v, rows_a, rows_b, tgt_v, sem,
             gsem_a, gsem_b, wsem_a, wsem_b):
    wid = lax.axis_index("c") * 16 + lax.axis_index("s")
    pltpu.sync_copy(idx3_hbm.at[wid], idx_v)
    bufs = (rows_a, rows_b)
    gsems = (gsem_a, gsem_b)
    wsems = (wsem_a, wsem_b)

    def _out(c):
        return g_hbm.at[pl.ds(wid * RPW + c * CHUNK, CHUNK), :]

    gd = pltpu.async_copy(t_hbm.at[idx_v.at[0]], bufs[0], gsems[0])
    wd = [None, None]
    for c in range(1, NCHUNK):
        b = c % 2
        if wd[b] is not None:
            wd[b].wait()
        gd_next = pltpu.async_copy(t_hbm.at[idx_v.at[c]], bufs[b], gsems[b])
        gd.wait()
        wd[1 - b] = pltpu.async_copy(bufs[1 - b], _out(c - 1), wsems[1 - b])
        gd = gd_next
    gd.wait()
    lastb = (NCHUNK - 1) % 2
    wd[lastb] = pltpu.async_copy(bufs[lastb], _out(NCHUNK - 1), wsems[lastb])
    wd[0].wait()
    wd[1].wait()
    pltpu.sync_copy(idxf_hbm.at[pl.ds(wid * RPW, RPW)], ev_v)
    ones16 = jnp.full((16,), 1.0, F32)

    def _memset(i, carry):
        tgt_v[pl.ds(pl.multiple_of(i * 16, 16), 16)] = ones16
        return carry

    lax.fori_loop(0, (BPW * KP) // 16, _memset, 0)
    zeros16 = jnp.zeros((16,), F32)
    lanes = lax.iota(jnp.int32, 16)
    for j in range(RPW // 16):
        e16 = ev_v[pl.ds(j * 16, 16)]
        bloc = (lanes + j * 16) // L
        kidx = jnp.where(e16 != 0, e16 - 1, jnp.int32(K))
        plsc.store_scatter(tgt_v, [bloc * KP + kidx], zeros16)
    pltpu.sync_copy(tgt_v, tgt_hbm.at[pl.ds(wid * (BPW * KP), BPW * KP)])


NWRD = K // 8


NWRDP = 128


def _pack_body(du_ref, p_ref):
    wi = jax.lax.broadcasted_iota(jnp.int32, (NWRDP, K), 0)
    ki = jax.lax.broadcasted_iota(jnp.int32, (NWRDP, K), 1)
    m = jnp.where(ki // 8 == wi, jnp.exp2((ki % 8).astype(F32)), 0.0)
    mb = m.astype(jnp.bfloat16)
    sel = jnp.where(du_ref[0] >= 0.5, 1.0, 0.0).astype(jnp.bfloat16)
    w = jnp.dot(mb, sel, preferred_element_type=F32)
    p_ref[0] = w.astype(jnp.bfloat16)


def _combine_body(pt_ref, g_ref, d_ref, o_ref):
    wi = jax.lax.broadcasted_iota(jnp.int32, (NWRDP, KP), 0)
    ki = jax.lax.broadcasted_iota(jnp.int32, (NWRDP, KP), 1)
    ex = jnp.where(ki // 8 == wi, jnp.exp2(-(ki % 8).astype(F32)), 0.0)
    exb = ex.astype(jnp.bfloat16)
    acc_lo = jnp.zeros((BBLK, KPH), F32)
    acc_hi = jnp.zeros((BBLK, K - KPH), F32)
    for l in range(L):
        g32 = g_ref[l]
        g_lo = jax.lax.bitcast_convert_type(g32 << 16, F32)
        g_hi = jax.lax.bitcast_convert_type(
            g32 & jnp.int32(-65536), F32)[:, :K - KPH]
        wf = pt_ref[:, l, :]
        e = jnp.dot(wf, exb, preferred_element_type=F32)
        bf = (e.astype(jnp.int32) & 1).astype(F32)
        acc_lo = acc_lo + g_lo * bf[:, :KPH]
        acc_hi = acc_hi + g_hi * bf[:, KPH:K]
    item_lo = acc_lo * 2.0
    item_hi = acc_hi * 2.0
    n2 = (jnp.sum(item_lo * item_lo, axis=1, keepdims=True)
          + jnp.sum(item_hi * item_hi, axis=1, keepdims=True))
    inv = 1.0 / jnp.maximum(jnp.sqrt(n2), 1e-5)
    o_ref[:, :KPH] = jnp.tanh(d_ref[:, :KPH] + item_lo * inv)
    o_ref[:, KPH:] = jnp.tanh(d_ref[:, KPH:K] + item_hi * inv)


def kernel(data, event_type, place_correlation, dropout_u, emb_table,
           W5, W1, W2, W3, W4, a, b, c, d):
    emb_pad = jnp.pad(emb_table, ((0, EP - (K + 1)), (0, 0)))
    w5t = jnp.pad(W5, ((0, KP - K), (0, 0))).T
    w1t = jnp.pad(W1, ((0, KP - K), (0, 0))).T
    w2t = jnp.pad(W2, ((0, KP - K), (0, 0))).T
    w3t = jnp.pad(W3, ((0, KP - K), (0, 0))).T
    w4t = jnp.pad(W4, ((0, KP - K), (0, 0))).T
    corr_pad = jnp.pad(place_correlation, ((1, EP - K - 1), (0, KP - K)))
    abcd = jnp.concatenate([a, b, c, d])
    idx_flat = event_type.reshape(-1)
    idx3 = event_type.T.reshape(NW, NCHUNK, CHUNK)

    t_tab, dense = pl.pallas_call(
        _tables_dense_body,
        out_shape=(jax.ShapeDtypeStruct((EP, KPH), jnp.int32),
                   jax.ShapeDtypeStruct((B, KP), F32)),
        in_specs=[
            pl.BlockSpec((EP, DIM), lambda: (0, 0)),
            pl.BlockSpec((DIM, KP), lambda: (0, 0)),
            pl.BlockSpec((EP, KP), lambda: (0, 0)),
            pl.BlockSpec((L, B, DIM), lambda: (0, 0, 0)),
            pl.BlockSpec((DIM, KP), lambda: (0, 0)),
            pl.BlockSpec((DIM, KP), lambda: (0, 0)),
            pl.BlockSpec((DIM, KP), lambda: (0, 0)),
            pl.BlockSpec((DIM, KP), lambda: (0, 0)),
            pl.BlockSpec(memory_space=pltpu.SMEM),
        ],
        out_specs=(pl.BlockSpec((EP, KPH), lambda: (0, 0)),
                   pl.BlockSpec((B, KP), lambda: (0, 0))),
    )(emb_pad, w5t, corr_pad, jnp.transpose(data, (1, 0, 2)),
      w1t, w2t, w3t, w4t, abcd)

    mesh = plsc.VectorSubcoreMesh(core_axis_name="c", subcore_axis_name="s",
                                  num_cores=2, num_subcores=16)
    sc = functools.partial(
        pl.kernel,
        compiler_params=pltpu.CompilerParams(needs_layout_passes=False),
        out_type=(jax.ShapeDtypeStruct((B * L, KPH), jnp.int32),
                  jax.ShapeDtypeStruct((B * KP,), F32)),
        mesh=mesh,
        scratch_types=[
            pltpu.VMEM((NCHUNK, CHUNK), jnp.int32),
            pltpu.VMEM((RPW,), jnp.int32),
            pltpu.VMEM((CHUNK, KPH), jnp.int32),
            pltpu.VMEM((CHUNK, KPH), jnp.int32),
            pltpu.VMEM((BPW * KP,), F32),
            pltpu.SemaphoreType.DMA,
            pltpu.SemaphoreType.DMA,
            pltpu.SemaphoreType.DMA,
            pltpu.SemaphoreType.DMA,
            pltpu.SemaphoreType.DMA,
        ],
    )(_sc_body)
    g_rows, tgt_flat = sc(idx3, idx_flat, t_tab)

    packed = pl.pallas_call(
        _pack_body,
        grid=(L,),
        out_shape=jax.ShapeDtypeStruct((L, NWRDP, B), jnp.bfloat16),
        in_specs=[pl.BlockSpec((1, K, B), lambda i: (i, 0, 0))],
        out_specs=pl.BlockSpec((1, NWRDP, B), lambda i: (i, 0, 0)),
    )(jnp.transpose(dropout_u, (1, 2, 0)))
    packed_t = jnp.transpose(packed, (2, 0, 1))

    out = pl.pallas_call(
        _combine_body,
        grid=(B // BBLK,),
        out_shape=jax.ShapeDtypeStruct((B, K), F32),
        in_specs=[
            pl.BlockSpec((BBLK, L, NWRDP), lambda i: (i, 0, 0)),
            pl.BlockSpec((L, BBLK, KPH), lambda i: (0, i, 0)),
            pl.BlockSpec((BBLK, KP), lambda i: (i, 0)),
        ],
        out_specs=pl.BlockSpec((BBLK, K), lambda i: (i, 0)),
    )(packed_t, g_rows.reshape(L, B, KPH), dense)

    target = tgt_flat.reshape(B, KP)[:, :K]
    return out, target


# speedup vs baseline: 1.1489x; 1.0640x over previous
import functools

import jax
import jax.numpy as jnp
from jax import lax
from jax.experimental import pallas as pl
from jax.experimental.pallas import tpu as pltpu
from jax.experimental.pallas import tpu_sc as plsc

B, L, DIM, K = 1024, 20, 128, 1000
KP = 1024
EP = 1008
NW = 32
RPW = (B * L) // NW
CHUNK = 64
NCHUNK = RPW // CHUNK
BPW = B // NW
BBLK = 64
KPH = 512
PBLK = 128
F32 = jnp.float32


_NT = (((1,), (1,)), ((), ()))


def _tables_dense_body(emb_ref, w5t_ref, corr_ref, data_ref, w1t_ref,
                       w2t_ref, w3t_ref, w4t_ref, abcd_ref, t_ref, d_ref):
    t = jnp.tanh(jax.lax.dot_general(emb_ref[...], w5t_ref[...], _NT,
                                     preferred_element_type=F32))
    tb = (t * corr_ref[...]).astype(jnp.bfloat16)
    lo = jax.lax.bitcast_convert_type(tb[:, :KPH], jnp.uint16).astype(jnp.uint32)
    hi = jax.lax.bitcast_convert_type(tb[:, KPH:], jnp.uint16).astype(jnp.uint32)
    t_ref[...] = jax.lax.bitcast_convert_type(lo | (hi << 16), jnp.int32)
    dmean = jnp.sum(data_ref[...], axis=0) * (1.0 / L)
    acc = jnp.zeros((B, K), F32)
    for i, wref in enumerate((w1t_ref, w2t_ref, w3t_ref, w4t_ref)):
        p = jax.lax.dot_general(dmean, wref[...], _NT,
                                preferred_element_type=F32)
        n = jnp.sqrt(jnp.sum(p * p, axis=1, keepdims=True))
        acc = acc + (p / jnp.maximum(n, 1e-5)) * abcd_ref[i]
    d_ref[...] = acc


def _sc_body(idx3_hbm, idxf_hbm, t_hbm, g_hbm, tgt_hbm,
             idx_v, ev_v, rows_a, rows_b, tgt_v, sem,
             gsem_a, gsem_b, wsem_a, wsem_b):
    wid = lax.axis_index("c") * 16 + lax.axis_index("s")
    pltpu.sync_copy(idx3_hbm.at[wid], idx_v)
    bufs = (rows_a, rows_b)
    gsems = (gsem_a, gsem_b)
    wsems = (wsem_a, wsem_b)

    def _out(c):
        return g_hbm.at[pl.ds(wid * RPW + c * CHUNK, CHUNK), :]

    gd = pltpu.async_copy(t_hbm.at[idx_v.at[0]], bufs[0], gsems[0])
    wd = [None, None]
    for c in range(1, NCHUNK):
        b = c % 2
        if wd[b] is not None:
            wd[b].wait()
        gd_next = pltpu.async_copy(t_hbm.at[idx_v.at[c]], bufs[b], gsems[b])
        gd.wait()
        wd[1 - b] = pltpu.async_copy(bufs[1 - b], _out(c - 1), wsems[1 - b])
        gd = gd_next
    gd.wait()
    lastb = (NCHUNK - 1) % 2
    wd[lastb] = pltpu.async_copy(bufs[lastb], _out(NCHUNK - 1), wsems[lastb])
    wd[0].wait()
    wd[1].wait()
    pltpu.sync_copy(idxf_hbm.at[pl.ds(wid * RPW, RPW)], ev_v)
    ones16 = jnp.full((16,), 1.0, F32)

    def _memset(i, carry):
        tgt_v[i // (KP // 16),
              pl.ds(pl.multiple_of((i % (KP // 16)) * 16, 16), 16)] = ones16
        return carry

    lax.fori_loop(0, (BPW * KP) // 16, _memset, 0)
    zeros16 = jnp.zeros((16,), F32)
    lanes = lax.iota(jnp.int32, 16)
    for j in range(RPW // 16):
        e16 = ev_v[pl.ds(j * 16, 16)]
        bloc = (lanes + j * 16) // L
        kidx = jnp.where(e16 != 0, e16 - 1, jnp.int32(K))
        plsc.store_scatter(tgt_v, [bloc, kidx], zeros16)
    pltpu.sync_copy(tgt_v, tgt_hbm.at[pl.ds(wid * BPW, BPW), :])


NWRD = K // 8


NWRDP = 128


def _pack_body(du_ref, p_ref):
    wi = jax.lax.broadcasted_iota(jnp.int32, (NWRDP, K), 0)
    ki = jax.lax.broadcasted_iota(jnp.int32, (NWRDP, K), 1)
    m = jnp.where(ki // 8 == wi, jnp.exp2((ki % 8).astype(F32)), 0.0)
    mb = m.astype(jnp.bfloat16)
    sel = jnp.where(du_ref[0] >= 0.5, 1.0, 0.0).astype(jnp.bfloat16)
    w = jnp.dot(mb, sel, preferred_element_type=F32)
    p_ref[0] = w.astype(jnp.bfloat16)


def _combine_body(pt_ref, g_ref, d_ref, o_ref):
    wi = jax.lax.broadcasted_iota(jnp.int32, (NWRDP, KP), 0)
    ki = jax.lax.broadcasted_iota(jnp.int32, (NWRDP, KP), 1)
    ex = jnp.where(ki // 8 == wi, jnp.exp2(-(ki % 8).astype(F32)), 0.0)
    exb = ex.astype(jnp.bfloat16)
    acc_lo = jnp.zeros((BBLK, KPH), F32)
    acc_hi = jnp.zeros((BBLK, K - KPH), F32)
    for l in range(L):
        g32 = g_ref[l]
        g_lo = jax.lax.bitcast_convert_type(g32 << 16, F32)
        g_hi = jax.lax.bitcast_convert_type(
            g32 & jnp.int32(-65536), F32)[:, :K - KPH]
        wf = pt_ref[:, l, :]
        e = jnp.dot(wf, exb, preferred_element_type=F32)
        bf = (e.astype(jnp.int32) & 1).astype(F32)
        acc_lo = acc_lo + g_lo * bf[:, :KPH]
        acc_hi = acc_hi + g_hi * bf[:, KPH:K]
    item_lo = acc_lo * 2.0
    item_hi = acc_hi * 2.0
    n2 = (jnp.sum(item_lo * item_lo, axis=1, keepdims=True)
          + jnp.sum(item_hi * item_hi, axis=1, keepdims=True))
    inv = 1.0 / jnp.maximum(jnp.sqrt(n2), 1e-5)
    o_ref[:, :KPH] = jnp.tanh(d_ref[:, :KPH] + item_lo * inv)
    o_ref[:, KPH:] = jnp.tanh(d_ref[:, KPH:] + item_hi * inv)


def kernel(data, event_type, place_correlation, dropout_u, emb_table,
           W5, W1, W2, W3, W4, a, b, c, d):
    emb_pad = jnp.pad(emb_table, ((0, EP - (K + 1)), (0, 0)))
    w5p = jnp.pad(W5, ((0, KP - K), (0, 0)))
    corr_pad = jnp.pad(place_correlation, ((1, EP - K - 1), (0, KP - K)))
    abcd = jnp.concatenate([a, b, c, d])
    idx_flat = event_type.reshape(-1)
    idx3 = event_type.T.reshape(NW, NCHUNK, CHUNK)

    t_tab, dense = pl.pallas_call(
        _tables_dense_body,
        out_shape=(jax.ShapeDtypeStruct((EP, KPH), jnp.int32),
                   jax.ShapeDtypeStruct((B, K), F32)),
        in_specs=[
            pl.BlockSpec((EP, DIM), lambda: (0, 0)),
            pl.BlockSpec((KP, DIM), lambda: (0, 0)),
            pl.BlockSpec((EP, KP), lambda: (0, 0)),
            pl.BlockSpec((L, B, DIM), lambda: (0, 0, 0)),
            pl.BlockSpec((K, DIM), lambda: (0, 0)),
            pl.BlockSpec((K, DIM), lambda: (0, 0)),
            pl.BlockSpec((K, DIM), lambda: (0, 0)),
            pl.BlockSpec((K, DIM), lambda: (0, 0)),
            pl.BlockSpec(memory_space=pltpu.SMEM),
        ],
        out_specs=(pl.BlockSpec((EP, KPH), lambda: (0, 0)),
                   pl.BlockSpec((B, K), lambda: (0, 0))),
    )(emb_pad, w5p, corr_pad, jnp.transpose(data, (1, 0, 2)),
      W1, W2, W3, W4, abcd)

    mesh = plsc.VectorSubcoreMesh(core_axis_name="c", subcore_axis_name="s",
                                  num_cores=2, num_subcores=16)
    sc = functools.partial(
        pl.kernel,
        compiler_params=pltpu.CompilerParams(needs_layout_passes=False),
        out_type=(jax.ShapeDtypeStruct((B * L, KPH), jnp.int32),
                  jax.ShapeDtypeStruct((B, KP), F32)),
        mesh=mesh,
        scratch_types=[
            pltpu.VMEM((NCHUNK, CHUNK), jnp.int32),
            pltpu.VMEM((RPW,), jnp.int32),
            pltpu.VMEM((CHUNK, KPH), jnp.int32),
            pltpu.VMEM((CHUNK, KPH), jnp.int32),
            pltpu.VMEM((BPW, KP), F32),
            pltpu.SemaphoreType.DMA,
            pltpu.SemaphoreType.DMA,
            pltpu.SemaphoreType.DMA,
            pltpu.SemaphoreType.DMA,
            pltpu.SemaphoreType.DMA,
        ],
    )(_sc_body)
    g_rows, tgt2d = sc(idx3, idx_flat, t_tab)

    packed = pl.pallas_call(
        _pack_body,
        grid=(L,),
        out_shape=jax.ShapeDtypeStruct((L, NWRDP, B), jnp.bfloat16),
        in_specs=[pl.BlockSpec((1, K, B), lambda i: (i, 0, 0))],
        out_specs=pl.BlockSpec((1, NWRDP, B), lambda i: (i, 0, 0)),
    )(jnp.transpose(dropout_u, (1, 2, 0)))
    packed_t = jnp.transpose(packed, (2, 0, 1))

    out = pl.pallas_call(
        _combine_body,
        grid=(B // BBLK,),
        out_shape=jax.ShapeDtypeStruct((B, K), F32),
        in_specs=[
            pl.BlockSpec((BBLK, L, NWRDP), lambda i: (i, 0, 0)),
            pl.BlockSpec((L, BBLK, KPH), lambda i: (0, i, 0)),
            pl.BlockSpec((BBLK, K), lambda i: (i, 0)),
        ],
        out_specs=pl.BlockSpec((BBLK, K), lambda i: (i, 0)),
    )(packed_t, g_rows.reshape(L, B, KPH), dense)

    target = tgt2d[:, :K]
    return out, target


# speedup vs baseline: 1.1560x; 1.0063x over previous
import functools

import jax
import jax.numpy as jnp
from jax import lax
from jax.experimental import pallas as pl
from jax.experimental.pallas import tpu as pltpu
from jax.experimental.pallas import tpu_sc as plsc

B, L, DIM, K = 1024, 20, 128, 1000
KP = 1024
EP = 1008
NW = 32
RPW = (B * L) // NW
CHUNK = 64
NCHUNK = RPW // CHUNK
BPW = B // NW
BBLK = 128
KPH = 512
PBLK = 128
F32 = jnp.float32


_NT = (((1,), (1,)), ((), ()))


def _tables_dense_body(emb_ref, w5t_ref, corr_ref, data_ref, w1t_ref,
                       w2t_ref, w3t_ref, w4t_ref, abcd_ref, t_ref, d_ref):
    t = jnp.tanh(jax.lax.dot_general(emb_ref[...], w5t_ref[...], _NT,
                                     preferred_element_type=F32))
    tb = (t * corr_ref[...]).astype(jnp.bfloat16)
    lo = jax.lax.bitcast_convert_type(tb[:, :KPH], jnp.uint16).astype(jnp.uint32)
    hi = jax.lax.bitcast_convert_type(tb[:, KPH:], jnp.uint16).astype(jnp.uint32)
    t_ref[...] = jax.lax.bitcast_convert_type(lo | (hi << 16), jnp.int32)
    dmean = jnp.sum(data_ref[...], axis=0) * (1.0 / L)
    acc = jnp.zeros((B, K), F32)
    for i, wref in enumerate((w1t_ref, w2t_ref, w3t_ref, w4t_ref)):
        p = jax.lax.dot_general(dmean, wref[...], _NT,
                                preferred_element_type=F32)
        n = jnp.sqrt(jnp.sum(p * p, axis=1, keepdims=True))
        acc = acc + (p / jnp.maximum(n, 1e-5)) * abcd_ref[i]
    d_ref[...] = acc


def _sc_body(idx3_hbm, idxf_hbm, t_hbm, g_hbm, tgt_hbm,
             idx_v, ev_v, rows_a, rows_b, tgt_v, sem,
             gsem_a, gsem_b, wsem_a, wsem_b):
    wid = lax.axis_index("c") * 16 + lax.axis_index("s")
    pltpu.sync_copy(idx3_hbm.at[wid], idx_v)
    bufs = (rows_a, rows_b)
    gsems = (gsem_a, gsem_b)
    wsems = (wsem_a, wsem_b)

    def _out(c):
        return g_hbm.at[pl.ds(wid * RPW + c * CHUNK, CHUNK), :]

    gd = pltpu.async_copy(t_hbm.at[idx_v.at[0]], bufs[0], gsems[0])
    wd = [None, None]
    for c in range(1, NCHUNK):
        b = c % 2
        if wd[b] is not None:
            wd[b].wait()
        gd_next = pltpu.async_copy(t_hbm.at[idx_v.at[c]], bufs[b], gsems[b])
        gd.wait()
        wd[1 - b] = pltpu.async_copy(bufs[1 - b], _out(c - 1), wsems[1 - b])
        gd = gd_next
    gd.wait()
    lastb = (NCHUNK - 1) % 2
    wd[lastb] = pltpu.async_copy(bufs[lastb], _out(NCHUNK - 1), wsems[lastb])
    wd[0].wait()
    wd[1].wait()
    pltpu.sync_copy(idxf_hbm.at[pl.ds(wid * RPW, RPW)], ev_v)
    ones16 = jnp.full((16,), 1.0, F32)

    def _memset(i, carry):
        tgt_v[i // (KP // 16),
              pl.ds(pl.multiple_of((i % (KP // 16)) * 16, 16), 16)] = ones16
        return carry

    lax.fori_loop(0, (BPW * KP) // 16, _memset, 0)
    zeros16 = jnp.zeros((16,), F32)
    lanes = lax.iota(jnp.int32, 16)
    for j in range(RPW // 16):
        e16 = ev_v[pl.ds(j * 16, 16)]
        bloc = (lanes + j * 16) // L
        kidx = jnp.where(e16 != 0, e16 - 1, jnp.int32(K))
        plsc.store_scatter(tgt_v, [bloc, kidx], zeros16)
    pltpu.sync_copy(tgt_v, tgt_hbm.at[pl.ds(wid * BPW, BPW), :])


NWRD = K // 8


NWRDP = 128


def _pack_body(du_ref, p_ref):
    wi = jax.lax.broadcasted_iota(jnp.int32, (NWRDP, K), 0)
    ki = jax.lax.broadcasted_iota(jnp.int32, (NWRDP, K), 1)
    m = jnp.where(ki // 8 == wi, jnp.exp2((ki % 8).astype(F32)), 0.0)
    mb = m.astype(jnp.bfloat16)
    sel = jnp.where(du_ref[0] >= 0.5, 1.0, 0.0).astype(jnp.bfloat16)
    w = jnp.dot(mb, sel, preferred_element_type=F32)
    p_ref[0] = w.astype(jnp.bfloat16)


def _combine_body(pt_ref, g_ref, d_ref, o_ref):
    wi = jax.lax.broadcasted_iota(jnp.int32, (NWRDP, KP), 0)
    ki = jax.lax.broadcasted_iota(jnp.int32, (NWRDP, KP), 1)
    ex = jnp.where(ki // 8 == wi, jnp.exp2(-(ki % 8).astype(F32)), 0.0)
    exb = ex.astype(jnp.bfloat16)
    acc_lo = jnp.zeros((BBLK, KPH), F32)
    acc_hi = jnp.zeros((BBLK, K - KPH), F32)
    for l in range(L):
        g32 = g_ref[l]
        g_lo = jax.lax.bitcast_convert_type(g32 << 16, F32)
        g_hi = jax.lax.bitcast_convert_type(
            g32 & jnp.int32(-65536), F32)[:, :K - KPH]
        wf = pt_ref[:, l, :]
        e = jnp.dot(wf, exb, preferred_element_type=F32)
        bf = (e.astype(jnp.int32) & 1).astype(F32)
        acc_lo = acc_lo + g_lo * bf[:, :KPH]
        acc_hi = acc_hi + g_hi * bf[:, KPH:K]
    item_lo = acc_lo * 2.0
    item_hi = acc_hi * 2.0
    n2 = (jnp.sum(item_lo * item_lo, axis=1, keepdims=True)
          + jnp.sum(item_hi * item_hi, axis=1, keepdims=True))
    inv = 1.0 / jnp.maximum(jnp.sqrt(n2), 1e-5)
    o_ref[:, :KPH] = jnp.tanh(d_ref[:, :KPH] + item_lo * inv)
    o_ref[:, KPH:] = jnp.tanh(d_ref[:, KPH:] + item_hi * inv)


def kernel(data, event_type, place_correlation, dropout_u, emb_table,
           W5, W1, W2, W3, W4, a, b, c, d):
    emb_pad = jnp.pad(emb_table, ((0, EP - (K + 1)), (0, 0)))
    w5p = jnp.pad(W5, ((0, KP - K), (0, 0)))
    corr_pad = jnp.pad(place_correlation, ((1, EP - K - 1), (0, KP - K)))
    abcd = jnp.concatenate([a, b, c, d])
    idx_flat = event_type.reshape(-1)
    idx3 = event_type.T.reshape(NW, NCHUNK, CHUNK)

    t_tab, dense = pl.pallas_call(
        _tables_dense_body,
        out_shape=(jax.ShapeDtypeStruct((EP, KPH), jnp.int32),
                   jax.ShapeDtypeStruct((B, K), F32)),
        in_specs=[
            pl.BlockSpec((EP, DIM), lambda: (0, 0)),
            pl.BlockSpec((KP, DIM), lambda: (0, 0)),
            pl.BlockSpec((EP, KP), lambda: (0, 0)),
            pl.BlockSpec((L, B, DIM), lambda: (0, 0, 0)),
            pl.BlockSpec((K, DIM), lambda: (0, 0)),
            pl.BlockSpec((K, DIM), lambda: (0, 0)),
            pl.BlockSpec((K, DIM), lambda: (0, 0)),
            pl.BlockSpec((K, DIM), lambda: (0, 0)),
            pl.BlockSpec(memory_space=pltpu.SMEM),
        ],
        out_specs=(pl.BlockSpec((EP, KPH), lambda: (0, 0)),
                   pl.BlockSpec((B, K), lambda: (0, 0))),
    )(emb_pad, w5p, corr_pad, jnp.transpose(data, (1, 0, 2)),
      W1, W2, W3, W4, abcd)

    mesh = plsc.VectorSubcoreMesh(core_axis_name="c", subcore_axis_name="s",
                                  num_cores=2, num_subcores=16)
    sc = functools.partial(
        pl.kernel,
        compiler_params=pltpu.CompilerParams(needs_layout_passes=False),
        out_type=(jax.ShapeDtypeStruct((B * L, KPH), jnp.int32),
                  jax.ShapeDtypeStruct((B, KP), F32)),
        mesh=mesh,
        scratch_types=[
            pltpu.VMEM((NCHUNK, CHUNK), jnp.int32),
            pltpu.VMEM((RPW,), jnp.int32),
            pltpu.VMEM((CHUNK, KPH), jnp.int32),
            pltpu.VMEM((CHUNK, KPH), jnp.int32),
            pltpu.VMEM((BPW, KP), F32),
            pltpu.SemaphoreType.DMA,
            pltpu.SemaphoreType.DMA,
            pltpu.SemaphoreType.DMA,
            pltpu.SemaphoreType.DMA,
            pltpu.SemaphoreType.DMA,
        ],
    )(_sc_body)
    g_rows, tgt2d = sc(idx3, idx_flat, t_tab)

    packed = pl.pallas_call(
        _pack_body,
        grid=(L,),
        out_shape=jax.ShapeDtypeStruct((L, NWRDP, B), jnp.bfloat16),
        in_specs=[pl.BlockSpec((1, K, B), lambda i: (i, 0, 0))],
        out_specs=pl.BlockSpec((1, NWRDP, B), lambda i: (i, 0, 0)),
    )(jnp.transpose(dropout_u, (1, 2, 0)))
    packed_t = jnp.transpose(packed, (2, 0, 1))

    out = pl.pallas_call(
        _combine_body,
        grid=(B // BBLK,),
        out_shape=jax.ShapeDtypeStruct((B, K), F32),
        in_specs=[
            pl.BlockSpec((BBLK, L, NWRDP), lambda i: (i, 0, 0)),
            pl.BlockSpec((L, BBLK, KPH), lambda i: (0, i, 0)),
            pl.BlockSpec((BBLK, K), lambda i: (i, 0)),
        ],
        out_specs=pl.BlockSpec((BBLK, K), lambda i: (i, 0)),
    )(packed_t, g_rows.reshape(L, B, KPH), dense)

    target = tgt2d[:, :K]
    return out, target


# speedup vs baseline: 1.1567x; 1.0006x over previous
import functools

import jax
import jax.numpy as jnp
from jax import lax
from jax.experimental import pallas as pl
from jax.experimental.pallas import tpu as pltpu
from jax.experimental.pallas import tpu_sc as plsc

B, L, DIM, K = 1024, 20, 128, 1000
KP = 1024
EP = 1008
NW = 32
RPW = (B * L) // NW
CHUNK = 64
NCHUNK = RPW // CHUNK
BPW = B // NW
BBLK = 128
KPH = 512
NWRDP = 128
F32 = jnp.float32

_NT = (((1,), (1,)), ((), ()))


def _tables_dense_body(emb_ref, w5t_ref, corr_ref, data_ref, w1t_ref,
                       w2t_ref, w3t_ref, w4t_ref, abcd_ref, t_ref, d_ref):
    t = jnp.tanh(jax.lax.dot_general(emb_ref[...], w5t_ref[...], _NT,
                                     preferred_element_type=F32))
    tb = (t * corr_ref[...]).astype(jnp.bfloat16)
    lo = jax.lax.bitcast_convert_type(tb[:, :KPH], jnp.uint16).astype(jnp.uint32)
    hi = jax.lax.bitcast_convert_type(tb[:, KPH:], jnp.uint16).astype(jnp.uint32)
    t_ref[...] = jax.lax.bitcast_convert_type(lo | (hi << 16), jnp.int32)
    dmean = jnp.sum(data_ref[...], axis=0) * (1.0 / L)
    acc = jnp.zeros((B, K), F32)
    for i, wref in enumerate((w1t_ref, w2t_ref, w3t_ref, w4t_ref)):
        p = jax.lax.dot_general(dmean, wref[...], _NT,
                                preferred_element_type=F32)
        n = jnp.sqrt(jnp.sum(p * p, axis=1, keepdims=True))
        acc = acc + (p / jnp.maximum(n, 1e-5)) * abcd_ref[i]
    d_ref[...] = acc


def _sc_body(idx3_hbm, idxf_hbm, t_hbm, g_hbm, tgt_hbm,
             idx_v, ev_v, rows_a, rows_b, tgt_v,
             gsem_a, gsem_b, wsem_a, wsem_b):
    wid = lax.axis_index("c") * 16 + lax.axis_index("s")
    pltpu.sync_copy(idx3_hbm.at[wid], idx_v)
    bufs = (rows_a, rows_b)
    gsems = (gsem_a, gsem_b)
    wsems = (wsem_a, wsem_b)

    def _out(c):
        return g_hbm.at[pl.ds(wid * RPW + c * CHUNK, CHUNK), :]

    gd = pltpu.async_copy(t_hbm.at[idx_v.at[0]], bufs[0], gsems[0])
    wd = [None, None]
    for c in range(1, NCHUNK):
        b = c % 2
        if wd[b] is not None:
            wd[b].wait()
        gd_next = pltpu.async_copy(t_hbm.at[idx_v.at[c]], bufs[b], gsems[b])
        gd.wait()
        wd[1 - b] = pltpu.async_copy(bufs[1 - b], _out(c - 1), wsems[1 - b])
        gd = gd_next
    gd.wait()
    lastb = (NCHUNK - 1) % 2
    wd[lastb] = pltpu.async_copy(bufs[lastb], _out(NCHUNK - 1), wsems[lastb])
    wd[0].wait()
    wd[1].wait()
    pltpu.sync_copy(idxf_hbm.at[pl.ds(wid * RPW, RPW)], ev_v)
    ones16 = jnp.full((16,), 1.0, F32)

    def _memset(i, carry):
        tgt_v[i // (KP // 16),
              pl.ds(pl.multiple_of((i % (KP // 16)) * 16, 16), 16)] = ones16
        return carry

    lax.fori_loop(0, (BPW * KP) // 16, _memset, 0)
    zeros16 = jnp.zeros((16,), F32)
    lanes = lax.iota(jnp.int32, 16)
    for j in range(RPW // 16):
        e16 = ev_v[pl.ds(j * 16, 16)]
        bloc = (lanes + j * 16) // L
        kidx = jnp.where(e16 != 0, e16 - 1, jnp.int32(K))
        plsc.store_scatter(tgt_v, [bloc, kidx], zeros16)
    pltpu.sync_copy(tgt_v, tgt_hbm.at[pl.ds(wid * BPW, BPW), :])


def _pack_body(du_ref, p_ref):
    wi = jax.lax.broadcasted_iota(jnp.int32, (NWRDP, K), 0)
    ki = jax.lax.broadcasted_iota(jnp.int32, (NWRDP, K), 1)
    m = jnp.where(ki // 8 == wi, jnp.exp2((ki % 8).astype(F32)), 0.0)
    mb = m.astype(jnp.bfloat16)
    sel = jnp.where(du_ref[0] >= 0.5, 1.0, 0.0).astype(jnp.bfloat16)
    w = jnp.dot(mb, sel, preferred_element_type=F32)
    p_ref[0] = w.astype(jnp.bfloat16)


def _combine_body(pt_ref, g_ref, d_ref, o_ref):
    wi = jax.lax.broadcasted_iota(jnp.int32, (NWRDP, KP), 0)
    ki = jax.lax.broadcasted_iota(jnp.int32, (NWRDP, KP), 1)
    ex = jnp.where(ki // 8 == wi, jnp.exp2(-(ki % 8).astype(F32)), 0.0)
    exb = ex.astype(jnp.bfloat16)
    acc_lo = jnp.zeros((BBLK, KPH), F32)
    acc_hi = jnp.zeros((BBLK, K - KPH), F32)
    for l in range(L):
        g32 = g_ref[l]
        g_lo = jax.lax.bitcast_convert_type(g32 << 16, F32)
        g_hi = jax.lax.bitcast_convert_type(
            g32 & jnp.int32(-65536), F32)[:, :K - KPH]
        wf = pt_ref[:, l, :]
        e = jnp.dot(wf, exb, preferred_element_type=F32)
        bf = (e.astype(jnp.int32) & 1).astype(F32)
        acc_lo = acc_lo + g_lo * bf[:, :KPH]
        acc_hi = acc_hi + g_hi * bf[:, KPH:K]
    item_lo = acc_lo * 2.0
    item_hi = acc_hi * 2.0
    n2 = (jnp.sum(item_lo * item_lo, axis=1, keepdims=True)
          + jnp.sum(item_hi * item_hi, axis=1, keepdims=True))
    inv = 1.0 / jnp.maximum(jnp.sqrt(n2), 1e-5)
    o_ref[:, :KPH] = jnp.tanh(d_ref[:, :KPH] + item_lo * inv)
    o_ref[:, KPH:] = jnp.tanh(d_ref[:, KPH:] + item_hi * inv)


def kernel(data, event_type, place_correlation, dropout_u, emb_table,
           W5, W1, W2, W3, W4, a, b, c, d):
    emb_pad = jnp.pad(emb_table, ((0, EP - (K + 1)), (0, 0)))
    w5p = jnp.pad(W5, ((0, KP - K), (0, 0)))
    corr_pad = jnp.pad(place_correlation, ((1, EP - K - 1), (0, KP - K)))
    abcd = jnp.concatenate([a, b, c, d])
    idx_flat = event_type.reshape(-1)
    idx3 = event_type.T.reshape(NW, NCHUNK, CHUNK)

    t_tab, dense = pl.pallas_call(
        _tables_dense_body,
        out_shape=(jax.ShapeDtypeStruct((EP, KPH), jnp.int32),
                   jax.ShapeDtypeStruct((B, K), F32)),
        in_specs=[
            pl.BlockSpec((EP, DIM), lambda: (0, 0)),
            pl.BlockSpec((KP, DIM), lambda: (0, 0)),
            pl.BlockSpec((EP, KP), lambda: (0, 0)),
            pl.BlockSpec((L, B, DIM), lambda: (0, 0, 0)),
            pl.BlockSpec((K, DIM), lambda: (0, 0)),
            pl.BlockSpec((K, DIM), lambda: (0, 0)),
            pl.BlockSpec((K, DIM), lambda: (0, 0)),
            pl.BlockSpec((K, DIM), lambda: (0, 0)),
            pl.BlockSpec(memory_space=pltpu.SMEM),
        ],
        out_specs=(pl.BlockSpec((EP, KPH), lambda: (0, 0)),
                   pl.BlockSpec((B, K), lambda: (0, 0))),
    )(emb_pad, w5p, corr_pad, jnp.transpose(data, (1, 0, 2)),
      W1, W2, W3, W4, abcd)

    mesh = plsc.VectorSubcoreMesh(core_axis_name="c", subcore_axis_name="s",
                                  num_cores=2, num_subcores=16)
    sc = functools.partial(
        pl.kernel,
        compiler_params=pltpu.CompilerParams(needs_layout_passes=False),
        out_type=(jax.ShapeDtypeStruct((B * L, KPH), jnp.int32),
                  jax.ShapeDtypeStruct((B, KP), F32)),
        mesh=mesh,
        scratch_types=[
            pltpu.VMEM((NCHUNK, CHUNK), jnp.int32),
            pltpu.VMEM((RPW,), jnp.int32),
            pltpu.VMEM((CHUNK, KPH), jnp.int32),
            pltpu.VMEM((CHUNK, KPH), jnp.int32),
            pltpu.VMEM((BPW, KP), F32),
            pltpu.SemaphoreType.DMA,
            pltpu.SemaphoreType.DMA,
            pltpu.SemaphoreType.DMA,
            pltpu.SemaphoreType.DMA,
        ],
    )(_sc_body)
    g_rows, tgt2d = sc(idx3, idx_flat, t_tab)

    packed = pl.pallas_call(
        _pack_body,
        grid=(L,),
        out_shape=jax.ShapeDtypeStruct((L, NWRDP, B), jnp.bfloat16),
        in_specs=[pl.BlockSpec((1, K, B), lambda i: (i, 0, 0))],
        out_specs=pl.BlockSpec((1, NWRDP, B), lambda i: (i, 0, 0)),
    )(jnp.transpose(dropout_u, (1, 2, 0)))
    packed_t = jnp.transpose(packed, (2, 0, 1))

    out = pl.pallas_call(
        _combine_body,
        grid=(B // BBLK,),
        out_shape=jax.ShapeDtypeStruct((B, K), F32),
        in_specs=[
            pl.BlockSpec((BBLK, L, NWRDP), lambda i: (i, 0, 0)),
            pl.BlockSpec((L, BBLK, KPH), lambda i: (0, i, 0)),
            pl.BlockSpec((BBLK, K), lambda i: (i, 0)),
        ],
        out_specs=pl.BlockSpec((BBLK, K), lambda i: (i, 0)),
    )(packed_t, g_rows.reshape(L, B, KPH), dense)

    target = tgt2d[:, :K]
    return out, target
